# Initial kernel scaffold; baseline (speedup 1.0000x reference)
import jax
import jax.numpy as jnp
from jax.experimental import pallas as pl


def kernel(x, W1, W2, W3, W4, W5, g1, b1, g2, b2, g3, b3, g4, b4, g5, b5):
    raise NotImplementedError("write your pallas kernel here")


# speedup vs baseline: 2.9826x; 2.9826x over previous
import functools

import jax
import jax.numpy as jnp
from jax import lax
from jax.experimental import pallas as pl
from jax.experimental.pallas import tpu as pltpu

KNN = 20
EPS = 1e-5
NEG = -3e38
POS = 3e38


def _dotT(a, b, prec):
    return lax.dot_general(a, b, (((1,), (1,)), ((), ())),
                           preferred_element_type=jnp.float32,
                           precision=prec)


def _knn_body(f_ref, e_ref, *, TN, N, C):
    nt = pl.program_id(1)
    ft = f_ref[0, pl.ds(nt * TN, TN), :]
    fb = f_ref[0]
    d = 2.0 * _dotT(ft, fb, lax.Precision.DEFAULT)
    d = d - _dotT(jnp.ones((TN, C), jnp.float32), fb * fb,
                  lax.Precision.HIGHEST)

    iota = lax.broadcasted_iota(jnp.int32, (TN, N), 1)

    def body(j, d):
        rm = jnp.max(d, axis=1, keepdims=True)
        amf = jnp.min(jnp.where(d == rm, iota, jnp.int32(N)),
                      axis=1, keepdims=True)
        hit = iota == amf
        d = jnp.where(hit, NEG, d)
        nbr = lax.dot_general(hit.astype(jnp.float32), fb,
                              (((1,), (0,)), ((), ())),
                              preferred_element_type=jnp.float32,
                              precision=lax.Precision.HIGHEST)
        e_ref[0, j] = nbr
        return d

    lax.fori_loop(0, KNN, body, d)


def _conv_body(e_ref, f_ref, w_ref, mx_ref, mn_ref, st_ref, *, TN, C, O):
    b = pl.program_id(0)
    nt = pl.program_id(1)

    @pl.when(jnp.logical_and(b == 0, nt == 0))
    def _():
        st_ref[...] = jnp.zeros_like(st_ref)

    ctr = f_ref[0]
    mx = jnp.full((TN, O), NEG, jnp.float32)
    mn = jnp.full((TN, O), POS, jnp.float32)
    s = jnp.zeros((1, O), jnp.float32)
    s2 = jnp.zeros((1, O), jnp.float32)
    for j in range(KNN):
        feat = jnp.concatenate([e_ref[0, j] - ctr, ctr], axis=1)
        y = jnp.dot(feat, w_ref[...], preferred_element_type=jnp.float32,
                    precision=lax.Precision.DEFAULT)
        mx = jnp.maximum(mx, y)
        mn = jnp.minimum(mn, y)
        s = s + jnp.sum(y, axis=0, keepdims=True)
        s2 = s2 + jnp.sum(y * y, axis=0, keepdims=True)
    mx_ref[0] = mx
    mn_ref[0] = mn
    st_ref[0:1, :] += s
    st_ref[1:2, :] += s2


def _apply_body(mx_ref, mn_ref, a_ref, c_ref, o_ref):
    a = a_ref[...]
    z = jnp.where(a >= 0, mx_ref[0], mn_ref[0])
    y = a * z + c_ref[...]
    o_ref[0] = jnp.where(y >= 0, y, 0.2 * y)


def _edge_stage(F, W, g, b, B, N, O, C, TN):
    NT = N // TN
    tile = pl.BlockSpec((1, TN, O), lambda bi, ni: (bi, ni, 0))

    E = pl.pallas_call(
        functools.partial(_knn_body, TN=TN, N=N, C=C),
        grid=(B, NT),
        in_specs=[pl.BlockSpec((1, N, C), lambda bi, ni: (bi, 0, 0))],
        out_specs=pl.BlockSpec((1, KNN, TN, C), lambda bi, ni: (bi, 0, ni, 0)),
        out_shape=jax.ShapeDtypeStruct((B, KNN, N, C), jnp.float32),
    )(F)

    mx, mn, st = pl.pallas_call(
        functools.partial(_conv_body, TN=TN, C=C, O=O),
        grid=(B, NT),
        in_specs=[pl.BlockSpec((1, KNN, TN, C), lambda bi, ni: (bi, 0, ni, 0)),
                  pl.BlockSpec((1, TN, C), lambda bi, ni: (bi, ni, 0)),
                  pl.BlockSpec((2 * C, O), lambda bi, ni: (0, 0))],
        out_specs=[tile, tile, pl.BlockSpec((8, O), lambda bi, ni: (0, 0))],
        out_shape=[jax.ShapeDtypeStruct((B, N, O), jnp.float32),
                   jax.ShapeDtypeStruct((B, N, O), jnp.float32),
                   jax.ShapeDtypeStruct((8, O), jnp.float32)],
    )(E, F, jnp.transpose(W))

    cnt = B * N * KNN
    mu = st[0] / cnt
    var = st[1] / cnt - mu * mu
    a = g / jnp.sqrt(var + EPS)
    c = b - a * mu

    F1 = pl.pallas_call(
        _apply_body,
        grid=(B, NT),
        in_specs=[tile, tile,
                  pl.BlockSpec((1, O), lambda bi, ni: (0, 0)),
                  pl.BlockSpec((1, O), lambda bi, ni: (0, 0))],
        out_specs=tile,
        out_shape=jax.ShapeDtypeStruct((B, N, O), jnp.float32),
    )(mx, mn, a[None, :], c[None, :])
    return F1


def _gram_body(f1_ref, f2_ref, f3_ref, f4_ref, g_ref, s_ref):
    i = pl.program_id(0)
    j = pl.program_id(1)

    @pl.when(jnp.logical_and(i == 0, j == 0))
    def _():
        g_ref[...] = jnp.zeros_like(g_ref)
        s_ref[...] = jnp.zeros_like(s_ref)

    cat = jnp.concatenate(
        [f1_ref[0], f2_ref[0], f3_ref[0], f4_ref[0]], axis=1)
    g_ref[...] += lax.dot_general(cat, cat, (((0,), (0,)), ((), ())),
                                  preferred_element_type=jnp.float32)
    s_ref[0:1, :] += jnp.sum(cat, axis=0, keepdims=True)


def _quad_body(g_ref, w_ref, q_ref):
    wg = jnp.dot(w_ref[...], g_ref[...], preferred_element_type=jnp.float32)
    q_ref[...] = jnp.sum(wg * w_ref[...], axis=1, keepdims=True)


def _head_body(f1_ref, f2_ref, f3_ref, f4_ref, w_ref, a_ref, c_ref, o_ref):
    ni = pl.program_id(1)
    cat = jnp.concatenate(
        [f1_ref[0], f2_ref[0], f3_ref[0], f4_ref[0]], axis=1)
    y = jnp.dot(cat, w_ref[...], preferred_element_type=jnp.float32,
                precision=lax.Precision.DEFAULT)
    y = a_ref[...] * y + c_ref[...]
    z = jnp.where(y >= 0, y, 0.2 * y)
    zmax = jnp.max(z, axis=0, keepdims=True)
    zsum = jnp.sum(z, axis=0, keepdims=True)

    @pl.when(ni == 0)
    def _():
        o_ref[0, 0:1, 0:512] = zmax
        o_ref[0, 0:1, 512:1024] = zsum

    @pl.when(ni != 0)
    def _():
        o_ref[0, 0:1, 0:512] = jnp.maximum(o_ref[0, 0:1, 0:512], zmax)
        o_ref[0, 0:1, 512:1024] += zsum


def _head(F1, F2, F3, F4, W5, g5, b5, B, N, TN):
    NT = N // TN

    def tiles(O):
        return pl.BlockSpec((1, TN, O), lambda bi, ni: (bi, ni, 0))

    G, S = pl.pallas_call(
        _gram_body,
        grid=(B, NT),
        in_specs=[tiles(64), tiles(64), tiles(128), tiles(256)],
        out_specs=[pl.BlockSpec((512, 512), lambda bi, ni: (0, 0)),
                   pl.BlockSpec((8, 512), lambda bi, ni: (0, 0))],
        out_shape=[jax.ShapeDtypeStruct((512, 512), jnp.float32),
                   jax.ShapeDtypeStruct((8, 512), jnp.float32)],
    )(F1, F2, F3, F4)

    q = pl.pallas_call(
        _quad_body,
        out_shape=jax.ShapeDtypeStruct((512, 1), jnp.float32),
    )(G, W5)[:, 0]

    cnt = B * N
    mu = jnp.dot(W5, S[0]) / cnt
    var = q / cnt - mu * mu
    a = g5 / jnp.sqrt(var + EPS)
    c = b5 - a * mu

    out = pl.pallas_call(
        _head_body,
        grid=(B, NT),
        in_specs=[tiles(64), tiles(64), tiles(128), tiles(256),
                  pl.BlockSpec((512, 512), lambda bi, ni: (0, 0)),
                  pl.BlockSpec((1, 512), lambda bi, ni: (0, 0)),
                  pl.BlockSpec((1, 512), lambda bi, ni: (0, 0))],
        out_specs=pl.BlockSpec((1, 8, 1024), lambda bi, ni: (bi, 0, 0)),
        out_shape=jax.ShapeDtypeStruct((B, 8, 1024), jnp.float32),
    )(F1, F2, F3, F4, jnp.transpose(W5), a[None, :], c[None, :])
    out = out[:, 0, :]
    return jnp.concatenate([out[:, :512], out[:, 512:] / N], axis=1)


def kernel(x, W1, W2, W3, W4, W5, g1, b1, g2, b2, g3, b3, g4, b4, g5, b5):
    B, C0, N = x.shape
    TN = 256
    F0 = jnp.transpose(x, (0, 2, 1))
    F1 = _edge_stage(F0, W1, g1, b1, B, N, 64, C0, TN)
    F2 = _edge_stage(F1, W2, g2, b2, B, N, 64, 64, TN)
    F3 = _edge_stage(F2, W3, g3, b3, B, N, 128, 64, TN)
    F4 = _edge_stage(F3, W4, g4, b4, B, N, 256, 128, TN)
    return _head(F1, F2, F3, F4, W5, g5, b5, B, N, TN)


# speedup vs baseline: 5.0386x; 1.6893x over previous
import functools

import jax
import jax.numpy as jnp
from jax import lax
from jax.experimental import pallas as pl
from jax.experimental.pallas import tpu as pltpu
from jax.experimental.pallas import tpu_sc as plsc

KNN = 20
_NC, _NS = 2, 16
_NW = _NC * _NS
EPS = 1e-5
NEG = -3e38
POS = 3e38


def _dotT(a, b, prec):
    return lax.dot_general(a, b, (((1,), (1,)), ((), ())),
                           preferred_element_type=jnp.float32,
                           precision=prec)


def _knn_body(f_ref, e_ref, *, TN, N, C):
    nt = pl.program_id(1)
    ft = f_ref[0, pl.ds(nt * TN, TN), :]
    fb = f_ref[0]
    d = 2.0 * _dotT(ft, fb, lax.Precision.DEFAULT)
    d = d - _dotT(jnp.ones((TN, C), jnp.float32), fb * fb,
                  lax.Precision.HIGHEST)

    iota = lax.broadcasted_iota(jnp.int32, (TN, N), 1)

    def body(j, d):
        rm = jnp.max(d, axis=1, keepdims=True)
        amf = jnp.min(jnp.where(d == rm, iota, jnp.int32(N)),
                      axis=1, keepdims=True)
        hit = iota == amf
        d = jnp.where(hit, NEG, d)
        nbr = lax.dot_general(hit.astype(jnp.float32), fb,
                              (((1,), (0,)), ((), ())),
                              preferred_element_type=jnp.float32,
                              precision=lax.Precision.HIGHEST)
        e_ref[0, j] = nbr
        return d

    lax.fori_loop(0, KNN, body, d)


def _knn_idx_body(f_ref, idx_ref, *, TN, N, C):
    b = pl.program_id(0)
    nt = pl.program_id(1)
    ft = f_ref[0, pl.ds(nt * TN, TN), :]
    fb = f_ref[0]
    d = 2.0 * _dotT(ft, fb, lax.Precision.DEFAULT)
    d = d - _dotT(jnp.ones((TN, C), jnp.float32), fb * fb,
                  lax.Precision.HIGHEST)

    iota = lax.broadcasted_iota(jnp.int32, (TN, N), 1)
    lane32 = lax.broadcasted_iota(jnp.int32, (TN, 32), 1)

    def body(j, carry):
        d, idx = carry
        rm = jnp.max(d, axis=1, keepdims=True)
        amf = jnp.min(jnp.where(d == rm, iota, jnp.int32(N)),
                      axis=1, keepdims=True)
        d = jnp.where(iota == amf, NEG, d)
        idx = jnp.where(lane32 == j, amf + b * N, idx)
        return d, idx

    _, idx = lax.fori_loop(0, KNN, body,
                           (d, jnp.zeros((TN, 32), jnp.int32)))
    idx_ref[0] = idx


def _sc_gather(idxg, f_flat, B, N, C):
    P = 128
    NCH = N // P
    TASKS = (B * NCH) // _NW
    mesh = plsc.VectorSubcoreMesh(core_axis_name="c", subcore_axis_name="s",
                                  num_cores=_NC, num_subcores=_NS)

    @functools.partial(
        pl.kernel, mesh=mesh,
        out_type=jax.ShapeDtypeStruct((B, KNN, N, C), jnp.float32),
        scratch_types=[pltpu.VMEM((P,), jnp.int32),
                       pltpu.VMEM((P, C), jnp.float32),
                       pltpu.SemaphoreType.DMA],
    )
    def k2(idx_hbm, f_hbm, e_hbm, listv, rows, sem):
        wid = lax.axis_index("s") * _NC + lax.axis_index("c")
        for t in range(TASKS):
            g = wid + _NW * t
            b = g // NCH
            n0 = (g % NCH) * P
            for j in range(KNN):
                pltpu.sync_copy(idx_hbm.at[b, j, pl.ds(n0, P)], listv)
                pltpu.async_copy(f_hbm.at[listv], rows, sem).wait()
                pltpu.sync_copy(rows, e_hbm.at[b, j, pl.ds(n0, P), :])

    return k2(idxg, f_flat)


def _conv_body(e_ref, f_ref, w_ref, mx_ref, mn_ref, st_ref, *, TN, C, O):
    b = pl.program_id(0)
    nt = pl.program_id(1)

    @pl.when(jnp.logical_and(b == 0, nt == 0))
    def _():
        st_ref[...] = jnp.zeros_like(st_ref)

    ctr = f_ref[0]
    mx = jnp.full((TN, O), NEG, jnp.float32)
    mn = jnp.full((TN, O), POS, jnp.float32)
    s = jnp.zeros((1, O), jnp.float32)
    s2 = jnp.zeros((1, O), jnp.float32)
    for j in range(KNN):
        feat = jnp.concatenate([e_ref[0, j, :, 0:C] - ctr, ctr],
                               axis=1)
        y = jnp.dot(feat, w_ref[...], preferred_element_type=jnp.float32,
                    precision=lax.Precision.DEFAULT)
        mx = jnp.maximum(mx, y)
        mn = jnp.minimum(mn, y)
        s = s + jnp.sum(y, axis=0, keepdims=True)
        s2 = s2 + jnp.sum(y * y, axis=0, keepdims=True)
    mx_ref[0] = mx
    mn_ref[0] = mn
    st_ref[0:1, :] += s
    st_ref[1:2, :] += s2


def _apply_body(mx_ref, mn_ref, a_ref, c_ref, o_ref):
    a = a_ref[...]
    z = jnp.where(a >= 0, mx_ref[0], mn_ref[0])
    y = a * z + c_ref[...]
    o_ref[0] = jnp.where(y >= 0, y, 0.2 * y)


def _edge_stage(F, W, g, b, B, N, O, C, TN):
    NT = N // TN
    tile = pl.BlockSpec((1, TN, O), lambda bi, ni: (bi, ni, 0))

    if C == 3:
        CP = C
        E = pl.pallas_call(
            functools.partial(_knn_body, TN=TN, N=N, C=C),
            grid=(B, NT),
            in_specs=[pl.BlockSpec((1, N, C), lambda bi, ni: (bi, 0, 0))],
            out_specs=pl.BlockSpec((1, KNN, TN, C),
                                   lambda bi, ni: (bi, 0, ni, 0)),
            out_shape=jax.ShapeDtypeStruct((B, KNN, N, C), jnp.float32),
        )(F)
    else:
        idx = pl.pallas_call(
            functools.partial(_knn_idx_body, TN=TN, N=N, C=C),
            grid=(B, NT),
            in_specs=[pl.BlockSpec((1, N, C), lambda bi, ni: (bi, 0, 0))],
            out_specs=pl.BlockSpec((1, TN, 32), lambda bi, ni: (bi, ni, 0)),
            out_shape=jax.ShapeDtypeStruct((B, N, 32), jnp.int32),
        )(F)
        idxT = jnp.transpose(idx, (0, 2, 1))[:, :KNN, :]
        CP = 128
        Fp = F if C == CP else jnp.pad(F, ((0, 0), (0, 0), (0, CP - C)))
        E = _sc_gather(idxT, jnp.reshape(Fp, (B * N, CP)), B, N, CP)

    mx, mn, st = pl.pallas_call(
        functools.partial(_conv_body, TN=TN, C=C, O=O),
        grid=(B, NT),
        in_specs=[pl.BlockSpec((1, KNN, TN, CP), lambda bi, ni: (bi, 0, ni, 0)),
                  pl.BlockSpec((1, TN, C), lambda bi, ni: (bi, ni, 0)),
                  pl.BlockSpec((2 * C, O), lambda bi, ni: (0, 0))],
        out_specs=[tile, tile, pl.BlockSpec((8, O), lambda bi, ni: (0, 0))],
        out_shape=[jax.ShapeDtypeStruct((B, N, O), jnp.float32),
                   jax.ShapeDtypeStruct((B, N, O), jnp.float32),
                   jax.ShapeDtypeStruct((8, O), jnp.float32)],
    )(E, F, jnp.transpose(W))

    cnt = B * N * KNN
    mu = st[0] / cnt
    var = st[1] / cnt - mu * mu
    a = g / jnp.sqrt(var + EPS)
    c = b - a * mu

    F1 = pl.pallas_call(
        _apply_body,
        grid=(B, NT),
        in_specs=[tile, tile,
                  pl.BlockSpec((1, O), lambda bi, ni: (0, 0)),
                  pl.BlockSpec((1, O), lambda bi, ni: (0, 0))],
        out_specs=tile,
        out_shape=jax.ShapeDtypeStruct((B, N, O), jnp.float32),
    )(mx, mn, a[None, :], c[None, :])
    return F1


def _gram_body(f1_ref, f2_ref, f3_ref, f4_ref, g_ref, s_ref):
    i = pl.program_id(0)
    j = pl.program_id(1)

    @pl.when(jnp.logical_and(i == 0, j == 0))
    def _():
        g_ref[...] = jnp.zeros_like(g_ref)
        s_ref[...] = jnp.zeros_like(s_ref)

    cat = jnp.concatenate(
        [f1_ref[0], f2_ref[0], f3_ref[0], f4_ref[0]], axis=1)
    g_ref[...] += lax.dot_general(cat, cat, (((0,), (0,)), ((), ())),
                                  preferred_element_type=jnp.float32)
    s_ref[0:1, :] += jnp.sum(cat, axis=0, keepdims=True)


def _quad_body(g_ref, w_ref, q_ref):
    wg = jnp.dot(w_ref[...], g_ref[...], preferred_element_type=jnp.float32)
    q_ref[...] = jnp.sum(wg * w_ref[...], axis=1, keepdims=True)


def _head_body(f1_ref, f2_ref, f3_ref, f4_ref, w_ref, a_ref, c_ref, o_ref):
    ni = pl.program_id(1)
    cat = jnp.concatenate(
        [f1_ref[0], f2_ref[0], f3_ref[0], f4_ref[0]], axis=1)
    y = jnp.dot(cat, w_ref[...], preferred_element_type=jnp.float32,
                precision=lax.Precision.DEFAULT)
    y = a_ref[...] * y + c_ref[...]
    z = jnp.where(y >= 0, y, 0.2 * y)
    zmax = jnp.max(z, axis=0, keepdims=True)
    zsum = jnp.sum(z, axis=0, keepdims=True)

    @pl.when(ni == 0)
    def _():
        o_ref[0, 0:1, 0:512] = zmax
        o_ref[0, 0:1, 512:1024] = zsum

    @pl.when(ni != 0)
    def _():
        o_ref[0, 0:1, 0:512] = jnp.maximum(o_ref[0, 0:1, 0:512], zmax)
        o_ref[0, 0:1, 512:1024] += zsum


def _head(F1, F2, F3, F4, W5, g5, b5, B, N, TN):
    NT = N // TN

    def tiles(O):
        return pl.BlockSpec((1, TN, O), lambda bi, ni: (bi, ni, 0))

    G, S = pl.pallas_call(
        _gram_body,
        grid=(B, NT),
        in_specs=[tiles(64), tiles(64), tiles(128), tiles(256)],
        out_specs=[pl.BlockSpec((512, 512), lambda bi, ni: (0, 0)),
                   pl.BlockSpec((8, 512), lambda bi, ni: (0, 0))],
        out_shape=[jax.ShapeDtypeStruct((512, 512), jnp.float32),
                   jax.ShapeDtypeStruct((8, 512), jnp.float32)],
    )(F1, F2, F3, F4)

    q = pl.pallas_call(
        _quad_body,
        out_shape=jax.ShapeDtypeStruct((512, 1), jnp.float32),
    )(G, W5)[:, 0]

    cnt = B * N
    mu = jnp.dot(W5, S[0]) / cnt
    var = q / cnt - mu * mu
    a = g5 / jnp.sqrt(var + EPS)
    c = b5 - a * mu

    out = pl.pallas_call(
        _head_body,
        grid=(B, NT),
        in_specs=[tiles(64), tiles(64), tiles(128), tiles(256),
                  pl.BlockSpec((512, 512), lambda bi, ni: (0, 0)),
                  pl.BlockSpec((1, 512), lambda bi, ni: (0, 0)),
                  pl.BlockSpec((1, 512), lambda bi, ni: (0, 0))],
        out_specs=pl.BlockSpec((1, 8, 1024), lambda bi, ni: (bi, 0, 0)),
        out_shape=jax.ShapeDtypeStruct((B, 8, 1024), jnp.float32),
    )(F1, F2, F3, F4, jnp.transpose(W5), a[None, :], c[None, :])
    out = out[:, 0, :]
    return jnp.concatenate([out[:, :512], out[:, 512:] / N], axis=1)


def kernel(x, W1, W2, W3, W4, W5, g1, b1, g2, b2, g3, b3, g4, b4, g5, b5):
    B, C0, N = x.shape
    TN = 256
    F0 = jnp.transpose(x, (0, 2, 1))
    F1 = _edge_stage(F0, W1, g1, b1, B, N, 64, C0, TN)
    F2 = _edge_stage(F1, W2, g2, b2, B, N, 64, 64, TN)
    F3 = _edge_stage(F2, W3, g3, b3, B, N, 128, 64, TN)
    F4 = _edge_stage(F3, W4, g4, b4, B, N, 256, 128, TN)
    return _head(F1, F2, F3, F4, W5, g5, b5, B, N, TN)


# speedup vs baseline: 6.5486x; 1.2997x over previous
import functools

import jax
import jax.numpy as jnp
from jax import lax
from jax.experimental import pallas as pl
from jax.experimental.pallas import tpu as pltpu
from jax.experimental.pallas import tpu_sc as plsc

KNN = 20
_NC, _NS = 2, 16
_NW = _NC * _NS
EPS = 1e-5
NEG = -3e38
POS = 3e38


def _dotT(a, b, prec):
    return lax.dot_general(a, b, (((1,), (1,)), ((), ())),
                           preferred_element_type=jnp.float32,
                           precision=prec)


def _knn_idx_body(f_ref, idx_ref, *, TN, N, C):
    b = pl.program_id(0)
    nt = pl.program_id(1)
    ft = f_ref[0, pl.ds(nt * TN, TN), :]
    fb = f_ref[0]
    d = 2.0 * _dotT(ft, fb, lax.Precision.DEFAULT)
    d = d - _dotT(jnp.ones((TN, C), jnp.float32), fb * fb,
                  lax.Precision.HIGHEST)

    iota = lax.broadcasted_iota(jnp.int32, (TN, N), 1)
    lane32 = lax.broadcasted_iota(jnp.int32, (TN, 32), 1)

    def body(j, carry):
        d, idx = carry
        rm = jnp.max(d, axis=1, keepdims=True)
        hit = d == rm
        amf = jnp.min(jnp.where(hit, iota, jnp.int32(N)),
                      axis=1, keepdims=True)
        d = jnp.where(iota == amf, NEG, d)
        idx = jnp.where(lane32 == j, amf + b * N, idx)
        return d, idx

    _, idx = lax.fori_loop(0, KNN, body,
                           (d, jnp.zeros((TN, 32), jnp.int32)))
    idx_ref[0] = idx


def _sc_gather(idxg, f_flat, B, N, C):
    P = 128
    NCH = N // P
    TASKS = (B * NCH) // _NW
    mesh = plsc.VectorSubcoreMesh(core_axis_name="c", subcore_axis_name="s",
                                  num_cores=_NC, num_subcores=_NS)

    @functools.partial(
        pl.kernel, mesh=mesh,
        out_type=jax.ShapeDtypeStruct((B, KNN, N, C), jnp.float32),
        scratch_types=[pltpu.VMEM((P,), jnp.int32),
                       pltpu.VMEM((P, C), jnp.float32),
                       pltpu.SemaphoreType.DMA],
    )
    def k2(idx_hbm, f_hbm, e_hbm, listv, rows, sem):
        wid = lax.axis_index("s") * _NC + lax.axis_index("c")
        for t in range(TASKS):
            g = wid + _NW * t
            b = g // NCH
            n0 = (g % NCH) * P
            for j in range(KNN):
                pltpu.sync_copy(idx_hbm.at[b, j, pl.ds(n0, P)], listv)
                pltpu.async_copy(f_hbm.at[listv], rows, sem).wait()
                pltpu.sync_copy(rows, e_hbm.at[b, j, pl.ds(n0, P), :])

    return k2(idxg, f_flat)


def _conv_body(e_ref, f_ref, w_ref, mx_ref, mn_ref, st_ref, *, TN, C, O):
    b = pl.program_id(0)
    nt = pl.program_id(1)

    @pl.when(jnp.logical_and(b == 0, nt == 0))
    def _():
        st_ref[...] = jnp.zeros_like(st_ref)

    ctr = f_ref[0]
    mx = jnp.full((TN, O), NEG, jnp.float32)
    mn = jnp.full((TN, O), POS, jnp.float32)
    s = jnp.zeros((1, O), jnp.float32)
    s2 = jnp.zeros((1, O), jnp.float32)
    for j in range(KNN):
        feat = jnp.concatenate([e_ref[0, j, :, 0:C] - ctr, ctr],
                               axis=1)
        y = jnp.dot(feat, w_ref[...], preferred_element_type=jnp.float32,
                    precision=lax.Precision.DEFAULT)
        mx = jnp.maximum(mx, y)
        mn = jnp.minimum(mn, y)
        s = s + jnp.sum(y, axis=0, keepdims=True)
        s2 = s2 + jnp.sum(y * y, axis=0, keepdims=True)
    mx_ref[0] = mx
    mn_ref[0] = mn
    st_ref[0:1, :] += s
    st_ref[1:2, :] += s2


def _apply_body(mx_ref, mn_ref, a_ref, c_ref, o_ref):
    a = a_ref[...]
    z = jnp.where(a >= 0, mx_ref[0], mn_ref[0])
    y = a * z + c_ref[...]
    o_ref[0] = jnp.where(y >= 0, y, 0.2 * y)


def _edge_stage(F, W, g, b, B, N, O, C, TN):
    NT = N // TN
    tile = pl.BlockSpec((1, TN, O), lambda bi, ni: (bi, ni, 0))

    idx = pl.pallas_call(
        functools.partial(_knn_idx_body, TN=TN, N=N, C=C),
        grid=(B, NT),
        in_specs=[pl.BlockSpec((1, N, C), lambda bi, ni: (bi, 0, 0))],
        out_specs=pl.BlockSpec((1, TN, 32), lambda bi, ni: (bi, ni, 0)),
        out_shape=jax.ShapeDtypeStruct((B, N, 32), jnp.int32),
    )(F)
    idxT = jnp.transpose(idx, (0, 2, 1))[:, :KNN, :]
    CP = 128
    Fp = F if C == CP else jnp.pad(F, ((0, 0), (0, 0), (0, CP - C)))
    E = _sc_gather(idxT, jnp.reshape(Fp, (B * N, CP)), B, N, CP)

    mx, mn, st = pl.pallas_call(
        functools.partial(_conv_body, TN=TN, C=C, O=O),
        grid=(B, NT),
        in_specs=[pl.BlockSpec((1, KNN, TN, CP), lambda bi, ni: (bi, 0, ni, 0)),
                  pl.BlockSpec((1, TN, C), lambda bi, ni: (bi, ni, 0)),
                  pl.BlockSpec((2 * C, O), lambda bi, ni: (0, 0))],
        out_specs=[tile, tile, pl.BlockSpec((8, O), lambda bi, ni: (0, 0))],
        out_shape=[jax.ShapeDtypeStruct((B, N, O), jnp.float32),
                   jax.ShapeDtypeStruct((B, N, O), jnp.float32),
                   jax.ShapeDtypeStruct((8, O), jnp.float32)],
    )(E, F, jnp.transpose(W))

    cnt = B * N * KNN
    mu = st[0] / cnt
    var = st[1] / cnt - mu * mu
    a = g / jnp.sqrt(var + EPS)
    c = b - a * mu

    F1 = pl.pallas_call(
        _apply_body,
        grid=(B, NT),
        in_specs=[tile, tile,
                  pl.BlockSpec((1, O), lambda bi, ni: (0, 0)),
                  pl.BlockSpec((1, O), lambda bi, ni: (0, 0))],
        out_specs=tile,
        out_shape=jax.ShapeDtypeStruct((B, N, O), jnp.float32),
    )(mx, mn, a[None, :], c[None, :])
    return F1


def _gram_body(f1_ref, f2_ref, f3_ref, f4_ref, g_ref, s_ref):
    i = pl.program_id(0)
    j = pl.program_id(1)

    @pl.when(jnp.logical_and(i == 0, j == 0))
    def _():
        g_ref[...] = jnp.zeros_like(g_ref)
        s_ref[...] = jnp.zeros_like(s_ref)

    cat = jnp.concatenate(
        [f1_ref[0], f2_ref[0], f3_ref[0], f4_ref[0]], axis=1)
    g_ref[...] += lax.dot_general(cat, cat, (((0,), (0,)), ((), ())),
                                  preferred_element_type=jnp.float32)
    s_ref[0:1, :] += jnp.sum(cat, axis=0, keepdims=True)


def _quad_body(g_ref, w_ref, q_ref):
    wg = jnp.dot(w_ref[...], g_ref[...], preferred_element_type=jnp.float32)
    q_ref[...] = jnp.sum(wg * w_ref[...], axis=1, keepdims=True)


def _head_body(f1_ref, f2_ref, f3_ref, f4_ref, w_ref, a_ref, c_ref, o_ref):
    ni = pl.program_id(1)
    cat = jnp.concatenate(
        [f1_ref[0], f2_ref[0], f3_ref[0], f4_ref[0]], axis=1)
    y = jnp.dot(cat, w_ref[...], preferred_element_type=jnp.float32,
                precision=lax.Precision.DEFAULT)
    y = a_ref[...] * y + c_ref[...]
    z = jnp.where(y >= 0, y, 0.2 * y)
    zmax = jnp.max(z, axis=0, keepdims=True)
    zsum = jnp.sum(z, axis=0, keepdims=True)

    @pl.when(ni == 0)
    def _():
        o_ref[0, 0:1, 0:512] = zmax
        o_ref[0, 0:1, 512:1024] = zsum

    @pl.when(ni != 0)
    def _():
        o_ref[0, 0:1, 0:512] = jnp.maximum(o_ref[0, 0:1, 0:512], zmax)
        o_ref[0, 0:1, 512:1024] += zsum


def _head(F1, F2, F3, F4, W5, g5, b5, B, N, TN):
    NT = N // TN

    def tiles(O):
        return pl.BlockSpec((1, TN, O), lambda bi, ni: (bi, ni, 0))

    G, S = pl.pallas_call(
        _gram_body,
        grid=(B, NT),
        in_specs=[tiles(64), tiles(64), tiles(128), tiles(256)],
        out_specs=[pl.BlockSpec((512, 512), lambda bi, ni: (0, 0)),
                   pl.BlockSpec((8, 512), lambda bi, ni: (0, 0))],
        out_shape=[jax.ShapeDtypeStruct((512, 512), jnp.float32),
                   jax.ShapeDtypeStruct((8, 512), jnp.float32)],
    )(F1, F2, F3, F4)

    q = pl.pallas_call(
        _quad_body,
        out_shape=jax.ShapeDtypeStruct((512, 1), jnp.float32),
    )(G, W5)[:, 0]

    cnt = B * N
    mu = jnp.dot(W5, S[0]) / cnt
    var = q / cnt - mu * mu
    a = g5 / jnp.sqrt(var + EPS)
    c = b5 - a * mu

    out = pl.pallas_call(
        _head_body,
        grid=(B, NT),
        in_specs=[tiles(64), tiles(64), tiles(128), tiles(256),
                  pl.BlockSpec((512, 512), lambda bi, ni: (0, 0)),
                  pl.BlockSpec((1, 512), lambda bi, ni: (0, 0)),
                  pl.BlockSpec((1, 512), lambda bi, ni: (0, 0))],
        out_specs=pl.BlockSpec((1, 8, 1024), lambda bi, ni: (bi, 0, 0)),
        out_shape=jax.ShapeDtypeStruct((B, 8, 1024), jnp.float32),
    )(F1, F2, F3, F4, jnp.transpose(W5), a[None, :], c[None, :])
    out = out[:, 0, :]
    return jnp.concatenate([out[:, :512], out[:, 512:] / N], axis=1)


def kernel(x, W1, W2, W3, W4, W5, g1, b1, g2, b2, g3, b3, g4, b4, g5, b5):
    B, C0, N = x.shape
    TN = 256
    F0 = jnp.transpose(x, (0, 2, 1))
    F1 = _edge_stage(F0, W1, g1, b1, B, N, 64, C0, TN)
    F2 = _edge_stage(F1, W2, g2, b2, B, N, 64, 64, TN)
    F3 = _edge_stage(F2, W3, g3, b3, B, N, 128, 64, TN)
    F4 = _edge_stage(F3, W4, g4, b4, B, N, 256, 128, TN)
    return _head(F1, F2, F3, F4, W5, g5, b5, B, N, TN)


# speedup vs baseline: 6.7392x; 1.0291x over previous
import functools

import jax
import jax.numpy as jnp
from jax import lax
from jax.experimental import pallas as pl
from jax.experimental.pallas import tpu as pltpu
from jax.experimental.pallas import tpu_sc as plsc

KNN = 20
_NC, _NS = 2, 16
_NW = _NC * _NS
EPS = 1e-5
NEG = -3e38
POS = 3e38


def _dotT(a, b, prec):
    return lax.dot_general(a, b, (((1,), (1,)), ((), ())),
                           preferred_element_type=jnp.float32,
                           precision=prec)


def _knn_idx_body(f_ref, idx_ref, *, TN, N, C):
    b = pl.program_id(0)
    nt = pl.program_id(1)
    ft = f_ref[0, pl.ds(nt * TN, TN), :]
    fb = f_ref[0]
    d = 2.0 * _dotT(ft, fb, lax.Precision.DEFAULT)
    d = d - _dotT(jnp.ones((TN, C), jnp.float32), fb * fb,
                  lax.Precision.HIGHEST)

    iota = lax.broadcasted_iota(jnp.int32, (TN, N), 1)
    lane32 = lax.broadcasted_iota(jnp.int32, (TN, 32), 1)

    def body(j, carry):
        d, idx = carry
        rm = jnp.max(d, axis=1, keepdims=True)
        hit = d == rm
        amf = jnp.min(jnp.where(hit, iota, jnp.int32(N)),
                      axis=1, keepdims=True)
        d = jnp.where(iota == amf, NEG, d)
        idx = jnp.where(lane32 == j, amf + b * N, idx)
        return d, idx

    _, idx = lax.fori_loop(0, KNN, body,
                           (d, jnp.zeros((TN, 32), jnp.int32)))
    idx_ref[0] = idx


def _sc_gather(idxg, f_flat, B, N, C):
    P = 128
    NCH = N // P
    TASKS = (B * NCH) // _NW
    mesh = plsc.VectorSubcoreMesh(core_axis_name="c", subcore_axis_name="s",
                                  num_cores=_NC, num_subcores=_NS)

    @functools.partial(
        pl.kernel, mesh=mesh,
        out_type=jax.ShapeDtypeStruct((B, KNN, N, C), jnp.float32),
        scratch_types=[pltpu.VMEM((P,), jnp.int32),
                       pltpu.VMEM((P, C), jnp.float32),
                       pltpu.SemaphoreType.DMA],
    )
    def k2(idx_hbm, f_hbm, e_hbm, listv, rows, sem):
        wid = lax.axis_index("s") * _NC + lax.axis_index("c")
        for t in range(TASKS):
            g = wid + _NW * t
            b = g // NCH
            n0 = (g % NCH) * P
            for j in range(KNN):
                pltpu.sync_copy(idx_hbm.at[b, j, pl.ds(n0, P)], listv)
                pltpu.async_copy(f_hbm.at[listv], rows, sem).wait()
                pltpu.sync_copy(rows, e_hbm.at[b, j, pl.ds(n0, P), :])

    return k2(idxg, f_flat)


def _conv_body(e_ref, f_ref, w_ref, mx_ref, mn_ref, st_ref, *, TN, C, O):
    b = pl.program_id(0)
    nt = pl.program_id(1)

    @pl.when(jnp.logical_and(b == 0, nt == 0))
    def _():
        st_ref[...] = jnp.zeros_like(st_ref)

    ctr = f_ref[0]
    mx = jnp.full((TN, O), NEG, jnp.float32)
    mn = jnp.full((TN, O), POS, jnp.float32)
    s = jnp.zeros((1, O), jnp.float32)
    s2 = jnp.zeros((1, O), jnp.float32)
    for j in range(KNN):
        feat = jnp.concatenate([e_ref[0, j, :, 0:C] - ctr, ctr],
                               axis=1)
        y = jnp.dot(feat, w_ref[...], preferred_element_type=jnp.float32,
                    precision=lax.Precision.DEFAULT)
        mx = jnp.maximum(mx, y)
        mn = jnp.minimum(mn, y)
        s = s + jnp.sum(y, axis=0, keepdims=True)
        s2 = s2 + jnp.sum(y * y, axis=0, keepdims=True)
    mx_ref[0] = mx
    mn_ref[0] = mn
    st_ref[0:1, :] += s
    st_ref[1:2, :] += s2


def _apply_body(mx_ref, mn_ref, a_ref, c_ref, o_ref):
    a = a_ref[...]
    z = jnp.where(a >= 0, mx_ref[0], mn_ref[0])
    y = a * z + c_ref[...]
    o_ref[0] = jnp.where(y >= 0, y, 0.2 * y)


def _edge_stage(F, W, g, b, B, N, O, C, TN):
    NT = N // TN
    tile = pl.BlockSpec((1, TN, O), lambda bi, ni: (bi, ni, 0))

    TNK = 512
    idx = pl.pallas_call(
        functools.partial(_knn_idx_body, TN=TNK, N=N, C=C),
        grid=(B, N // TNK),
        in_specs=[pl.BlockSpec((1, N, C), lambda bi, ni: (bi, 0, 0))],
        out_specs=pl.BlockSpec((1, TNK, 32), lambda bi, ni: (bi, ni, 0)),
        out_shape=jax.ShapeDtypeStruct((B, N, 32), jnp.int32),
    )(F)
    idxT = jnp.transpose(idx, (0, 2, 1))[:, :KNN, :]
    CP = 128
    Fp = F if C == CP else jnp.pad(F, ((0, 0), (0, 0), (0, CP - C)))
    E = _sc_gather(idxT, jnp.reshape(Fp, (B * N, CP)), B, N, CP)

    mx, mn, st = pl.pallas_call(
        functools.partial(_conv_body, TN=TN, C=C, O=O),
        grid=(B, NT),
        in_specs=[pl.BlockSpec((1, KNN, TN, CP), lambda bi, ni: (bi, 0, ni, 0)),
                  pl.BlockSpec((1, TN, C), lambda bi, ni: (bi, ni, 0)),
                  pl.BlockSpec((2 * C, O), lambda bi, ni: (0, 0))],
        out_specs=[tile, tile, pl.BlockSpec((8, O), lambda bi, ni: (0, 0))],
        out_shape=[jax.ShapeDtypeStruct((B, N, O), jnp.float32),
                   jax.ShapeDtypeStruct((B, N, O), jnp.float32),
                   jax.ShapeDtypeStruct((8, O), jnp.float32)],
    )(E, F, jnp.transpose(W))

    cnt = B * N * KNN
    mu = st[0] / cnt
    var = st[1] / cnt - mu * mu
    a = g / jnp.sqrt(var + EPS)
    c = b - a * mu

    F1 = pl.pallas_call(
        _apply_body,
        grid=(B, NT),
        in_specs=[tile, tile,
                  pl.BlockSpec((1, O), lambda bi, ni: (0, 0)),
                  pl.BlockSpec((1, O), lambda bi, ni: (0, 0))],
        out_specs=tile,
        out_shape=jax.ShapeDtypeStruct((B, N, O), jnp.float32),
    )(mx, mn, a[None, :], c[None, :])
    return F1


def _gram_body(f1_ref, f2_ref, f3_ref, f4_ref, g_ref, s_ref):
    i = pl.program_id(0)
    j = pl.program_id(1)

    @pl.when(jnp.logical_and(i == 0, j == 0))
    def _():
        g_ref[...] = jnp.zeros_like(g_ref)
        s_ref[...] = jnp.zeros_like(s_ref)

    cat = jnp.concatenate(
        [f1_ref[0], f2_ref[0], f3_ref[0], f4_ref[0]], axis=1)
    g_ref[...] += lax.dot_general(cat, cat, (((0,), (0,)), ((), ())),
                                  preferred_element_type=jnp.float32)
    s_ref[0:1, :] += jnp.sum(cat, axis=0, keepdims=True)


def _quad_body(g_ref, w_ref, q_ref):
    wg = jnp.dot(w_ref[...], g_ref[...], preferred_element_type=jnp.float32)
    q_ref[...] = jnp.sum(wg * w_ref[...], axis=1, keepdims=True)


def _head_body(f1_ref, f2_ref, f3_ref, f4_ref, w_ref, a_ref, c_ref, o_ref):
    ni = pl.program_id(1)
    cat = jnp.concatenate(
        [f1_ref[0], f2_ref[0], f3_ref[0], f4_ref[0]], axis=1)
    y = jnp.dot(cat, w_ref[...], preferred_element_type=jnp.float32,
                precision=lax.Precision.DEFAULT)
    y = a_ref[...] * y + c_ref[...]
    z = jnp.where(y >= 0, y, 0.2 * y)
    zmax = jnp.max(z, axis=0, keepdims=True)
    zsum = jnp.sum(z, axis=0, keepdims=True)

    @pl.when(ni == 0)
    def _():
        o_ref[0, 0:1, 0:512] = zmax
        o_ref[0, 0:1, 512:1024] = zsum

    @pl.when(ni != 0)
    def _():
        o_ref[0, 0:1, 0:512] = jnp.maximum(o_ref[0, 0:1, 0:512], zmax)
        o_ref[0, 0:1, 512:1024] += zsum


def _head(F1, F2, F3, F4, W5, g5, b5, B, N, TN):
    NT = N // TN

    def tiles(O):
        return pl.BlockSpec((1, TN, O), lambda bi, ni: (bi, ni, 0))

    G, S = pl.pallas_call(
        _gram_body,
        grid=(B, NT),
        in_specs=[tiles(64), tiles(64), tiles(128), tiles(256)],
        out_specs=[pl.BlockSpec((512, 512), lambda bi, ni: (0, 0)),
                   pl.BlockSpec((8, 512), lambda bi, ni: (0, 0))],
        out_shape=[jax.ShapeDtypeStruct((512, 512), jnp.float32),
                   jax.ShapeDtypeStruct((8, 512), jnp.float32)],
    )(F1, F2, F3, F4)

    q = pl.pallas_call(
        _quad_body,
        out_shape=jax.ShapeDtypeStruct((512, 1), jnp.float32),
    )(G, W5)[:, 0]

    cnt = B * N
    mu = jnp.dot(W5, S[0]) / cnt
    var = q / cnt - mu * mu
    a = g5 / jnp.sqrt(var + EPS)
    c = b5 - a * mu

    out = pl.pallas_call(
        _head_body,
        grid=(B, NT),
        in_specs=[tiles(64), tiles(64), tiles(128), tiles(256),
                  pl.BlockSpec((512, 512), lambda bi, ni: (0, 0)),
                  pl.BlockSpec((1, 512), lambda bi, ni: (0, 0)),
                  pl.BlockSpec((1, 512), lambda bi, ni: (0, 0))],
        out_specs=pl.BlockSpec((1, 8, 1024), lambda bi, ni: (bi, 0, 0)),
        out_shape=jax.ShapeDtypeStruct((B, 8, 1024), jnp.float32),
    )(F1, F2, F3, F4, jnp.transpose(W5), a[None, :], c[None, :])
    out = out[:, 0, :]
    return jnp.concatenate([out[:, :512], out[:, 512:] / N], axis=1)


def kernel(x, W1, W2, W3, W4, W5, g1, b1, g2, b2, g3, b3, g4, b4, g5, b5):
    B, C0, N = x.shape
    TN = 256
    F0 = jnp.transpose(x, (0, 2, 1))
    F1 = _edge_stage(F0, W1, g1, b1, B, N, 64, C0, TN)
    F2 = _edge_stage(F1, W2, g2, b2, B, N, 64, 64, TN)
    F3 = _edge_stage(F2, W3, g3, b3, B, N, 128, 64, TN)
    F4 = _edge_stage(F3, W4, g4, b4, B, N, 256, 128, TN)
    return _head(F1, F2, F3, F4, W5, g5, b5, B, N, TN)


# speedup vs baseline: 6.7633x; 1.0036x over previous
import functools

import jax
import jax.numpy as jnp
from jax import lax
from jax.experimental import pallas as pl
from jax.experimental.pallas import tpu as pltpu
from jax.experimental.pallas import tpu_sc as plsc

KNN = 20
_NC, _NS = 2, 16
_NW = _NC * _NS
EPS = 1e-5
NEG = -3e38
POS = 3e38


def _dotT(a, b, prec):
    return lax.dot_general(a, b, (((1,), (1,)), ((), ())),
                           preferred_element_type=jnp.float32,
                           precision=prec)


def _knn_idx_body(f_ref, idx_ref, *, TN, N, C):
    b = pl.program_id(0)
    nt = pl.program_id(1)
    ft = f_ref[0, pl.ds(nt * TN, TN), :]
    fb = f_ref[0]
    d = 2.0 * _dotT(ft, fb, lax.Precision.DEFAULT)
    d = d - _dotT(jnp.ones((TN, C), jnp.float32), fb * fb,
                  lax.Precision.HIGHEST)

    iota = lax.broadcasted_iota(jnp.int32, (TN, N), 1)
    lane32 = lax.broadcasted_iota(jnp.int32, (TN, 32), 1)

    def body(j, carry):
        d, idx = carry
        rm = jnp.max(d, axis=1, keepdims=True)
        hit = d == rm
        amf = jnp.min(jnp.where(hit, iota, jnp.int32(N)),
                      axis=1, keepdims=True)
        d = jnp.where(iota == amf, NEG, d)
        idx = jnp.where(lane32 == j, amf + b * N, idx)
        return d, idx

    _, idx = lax.fori_loop(0, KNN, body,
                           (d, jnp.zeros((TN, 32), jnp.int32)))
    idx_ref[0] = idx


def _sc_gather(idxg, f_flat, B, N, C):
    P = 128
    NCH = N // P
    TASKS = (B * NCH) // _NW
    mesh = plsc.VectorSubcoreMesh(core_axis_name="c", subcore_axis_name="s",
                                  num_cores=_NC, num_subcores=_NS)

    @functools.partial(
        pl.kernel, mesh=mesh,
        out_type=jax.ShapeDtypeStruct((B, KNN, N, C), jnp.float32),
        scratch_types=[pltpu.VMEM((P,), jnp.int32),
                       pltpu.VMEM((P, C), jnp.float32),
                       pltpu.SemaphoreType.DMA],
    )
    def k2(idx_hbm, f_hbm, e_hbm, listv, rows, sem):
        wid = lax.axis_index("s") * _NC + lax.axis_index("c")
        for t in range(TASKS):
            g = wid + _NW * t
            b = g // NCH
            n0 = (g % NCH) * P
            for j in range(KNN):
                pltpu.sync_copy(idx_hbm.at[b, j, pl.ds(n0, P)], listv)
                pltpu.async_copy(f_hbm.at[listv], rows, sem).wait()
                pltpu.sync_copy(rows, e_hbm.at[b, j, pl.ds(n0, P), :])

    return k2(idxg, f_flat)


def _conv_body(e_ref, f_ref, w_ref, mx_ref, mn_ref, st_ref, *, TN, C, O):
    b = pl.program_id(0)
    nt = pl.program_id(1)

    @pl.when(jnp.logical_and(b == 0, nt == 0))
    def _():
        st_ref[...] = jnp.zeros_like(st_ref)

    ctr = f_ref[0]
    mx = jnp.full((TN, O), NEG, jnp.float32)
    mn = jnp.full((TN, O), POS, jnp.float32)
    s = jnp.zeros((1, O), jnp.float32)
    s2 = jnp.zeros((1, O), jnp.float32)
    for j in range(KNN):
        feat = jnp.concatenate([e_ref[0, j, :, 0:C] - ctr, ctr],
                               axis=1)
        y = jnp.dot(feat, w_ref[...], preferred_element_type=jnp.float32,
                    precision=lax.Precision.DEFAULT)
        mx = jnp.maximum(mx, y)
        mn = jnp.minimum(mn, y)
        s = s + jnp.sum(y, axis=0, keepdims=True)
        s2 = s2 + jnp.sum(y * y, axis=0, keepdims=True)
    mx_ref[0] = mx
    mn_ref[0] = mn
    st_ref[0:1, :] += s
    st_ref[1:2, :] += s2


def _apply_body(mx_ref, mn_ref, a_ref, c_ref, o_ref):
    a = a_ref[...]
    z = jnp.where(a >= 0, mx_ref[0], mn_ref[0])
    y = a * z + c_ref[...]
    o_ref[0] = jnp.where(y >= 0, y, 0.2 * y)


def _edge_stage(F, W, g, b, B, N, O, C, TN):
    NT = N // TN
    tile = pl.BlockSpec((1, TN, O), lambda bi, ni: (bi, ni, 0))

    TNK = 1024
    idx = pl.pallas_call(
        functools.partial(_knn_idx_body, TN=TNK, N=N, C=C),
        grid=(B, N // TNK),
        in_specs=[pl.BlockSpec((1, N, C), lambda bi, ni: (bi, 0, 0))],
        out_specs=pl.BlockSpec((1, TNK, 32), lambda bi, ni: (bi, ni, 0)),
        out_shape=jax.ShapeDtypeStruct((B, N, 32), jnp.int32),
    )(F)
    idxT = jnp.transpose(idx, (0, 2, 1))[:, :KNN, :]
    CP = 128
    Fp = F if C == CP else jnp.pad(F, ((0, 0), (0, 0), (0, CP - C)))
    E = _sc_gather(idxT, jnp.reshape(Fp, (B * N, CP)), B, N, CP)

    mx, mn, st = pl.pallas_call(
        functools.partial(_conv_body, TN=TN, C=C, O=O),
        grid=(B, NT),
        in_specs=[pl.BlockSpec((1, KNN, TN, CP), lambda bi, ni: (bi, 0, ni, 0)),
                  pl.BlockSpec((1, TN, C), lambda bi, ni: (bi, ni, 0)),
                  pl.BlockSpec((2 * C, O), lambda bi, ni: (0, 0))],
        out_specs=[tile, tile, pl.BlockSpec((8, O), lambda bi, ni: (0, 0))],
        out_shape=[jax.ShapeDtypeStruct((B, N, O), jnp.float32),
                   jax.ShapeDtypeStruct((B, N, O), jnp.float32),
                   jax.ShapeDtypeStruct((8, O), jnp.float32)],
    )(E, F, jnp.transpose(W))

    cnt = B * N * KNN
    mu = st[0] / cnt
    var = st[1] / cnt - mu * mu
    a = g / jnp.sqrt(var + EPS)
    c = b - a * mu

    F1 = pl.pallas_call(
        _apply_body,
        grid=(B, NT),
        in_specs=[tile, tile,
                  pl.BlockSpec((1, O), lambda bi, ni: (0, 0)),
                  pl.BlockSpec((1, O), lambda bi, ni: (0, 0))],
        out_specs=tile,
        out_shape=jax.ShapeDtypeStruct((B, N, O), jnp.float32),
    )(mx, mn, a[None, :], c[None, :])
    return F1


def _gram_body(f1_ref, f2_ref, f3_ref, f4_ref, g_ref, s_ref):
    i = pl.program_id(0)
    j = pl.program_id(1)

    @pl.when(jnp.logical_and(i == 0, j == 0))
    def _():
        g_ref[...] = jnp.zeros_like(g_ref)
        s_ref[...] = jnp.zeros_like(s_ref)

    cat = jnp.concatenate(
        [f1_ref[0], f2_ref[0], f3_ref[0], f4_ref[0]], axis=1)
    g_ref[...] += lax.dot_general(cat, cat, (((0,), (0,)), ((), ())),
                                  preferred_element_type=jnp.float32)
    s_ref[0:1, :] += jnp.sum(cat, axis=0, keepdims=True)


def _quad_body(g_ref, w_ref, q_ref):
    wg = jnp.dot(w_ref[...], g_ref[...], preferred_element_type=jnp.float32)
    q_ref[...] = jnp.sum(wg * w_ref[...], axis=1, keepdims=True)


def _head_body(f1_ref, f2_ref, f3_ref, f4_ref, w_ref, a_ref, c_ref, o_ref):
    ni = pl.program_id(1)
    cat = jnp.concatenate(
        [f1_ref[0], f2_ref[0], f3_ref[0], f4_ref[0]], axis=1)
    y = jnp.dot(cat, w_ref[...], preferred_element_type=jnp.float32,
                precision=lax.Precision.DEFAULT)
    y = a_ref[...] * y + c_ref[...]
    z = jnp.where(y >= 0, y, 0.2 * y)
    zmax = jnp.max(z, axis=0, keepdims=True)
    zsum = jnp.sum(z, axis=0, keepdims=True)

    @pl.when(ni == 0)
    def _():
        o_ref[0, 0:1, 0:512] = zmax
        o_ref[0, 0:1, 512:1024] = zsum

    @pl.when(ni != 0)
    def _():
        o_ref[0, 0:1, 0:512] = jnp.maximum(o_ref[0, 0:1, 0:512], zmax)
        o_ref[0, 0:1, 512:1024] += zsum


def _head(F1, F2, F3, F4, W5, g5, b5, B, N, TN):
    NT = N // TN

    def tiles(O):
        return pl.BlockSpec((1, TN, O), lambda bi, ni: (bi, ni, 0))

    G, S = pl.pallas_call(
        _gram_body,
        grid=(B, NT),
        in_specs=[tiles(64), tiles(64), tiles(128), tiles(256)],
        out_specs=[pl.BlockSpec((512, 512), lambda bi, ni: (0, 0)),
                   pl.BlockSpec((8, 512), lambda bi, ni: (0, 0))],
        out_shape=[jax.ShapeDtypeStruct((512, 512), jnp.float32),
                   jax.ShapeDtypeStruct((8, 512), jnp.float32)],
    )(F1, F2, F3, F4)

    q = pl.pallas_call(
        _quad_body,
        out_shape=jax.ShapeDtypeStruct((512, 1), jnp.float32),
    )(G, W5)[:, 0]

    cnt = B * N
    mu = jnp.dot(W5, S[0]) / cnt
    var = q / cnt - mu * mu
    a = g5 / jnp.sqrt(var + EPS)
    c = b5 - a * mu

    out = pl.pallas_call(
        _head_body,
        grid=(B, NT),
        in_specs=[tiles(64), tiles(64), tiles(128), tiles(256),
                  pl.BlockSpec((512, 512), lambda bi, ni: (0, 0)),
                  pl.BlockSpec((1, 512), lambda bi, ni: (0, 0)),
                  pl.BlockSpec((1, 512), lambda bi, ni: (0, 0))],
        out_specs=pl.BlockSpec((1, 8, 1024), lambda bi, ni: (bi, 0, 0)),
        out_shape=jax.ShapeDtypeStruct((B, 8, 1024), jnp.float32),
    )(F1, F2, F3, F4, jnp.transpose(W5), a[None, :], c[None, :])
    out = out[:, 0, :]
    return jnp.concatenate([out[:, :512], out[:, 512:] / N], axis=1)


def kernel(x, W1, W2, W3, W4, W5, g1, b1, g2, b2, g3, b3, g4, b4, g5, b5):
    B, C0, N = x.shape
    TN = 256
    F0 = jnp.transpose(x, (0, 2, 1))
    F1 = _edge_stage(F0, W1, g1, b1, B, N, 64, C0, TN)
    F2 = _edge_stage(F1, W2, g2, b2, B, N, 64, 64, TN)
    F3 = _edge_stage(F2, W3, g3, b3, B, N, 128, 64, TN)
    F4 = _edge_stage(F3, W4, g4, b4, B, N, 256, 128, TN)
    return _head(F1, F2, F3, F4, W5, g5, b5, B, N, TN)


# speedup vs baseline: 6.8325x; 1.0102x over previous
import functools

import jax
import jax.numpy as jnp
from jax import lax
from jax.experimental import pallas as pl
from jax.experimental.pallas import tpu as pltpu
from jax.experimental.pallas import tpu_sc as plsc

KNN = 20
_NC, _NS = 2, 16
_NW = _NC * _NS
EPS = 1e-5
NEG = -3e38
POS = 3e38


def _dotT(a, b, prec):
    return lax.dot_general(a, b, (((1,), (1,)), ((), ())),
                           preferred_element_type=jnp.float32,
                           precision=prec)


def _knn_idx_body(f_ref, idx_ref, *, TN, N, C):
    b = pl.program_id(0)
    nt = pl.program_id(1)
    ft = f_ref[0, pl.ds(nt * TN, TN), :]
    fb = f_ref[0]
    d = 2.0 * _dotT(ft, fb, lax.Precision.DEFAULT)
    d = d - _dotT(jnp.ones((TN, C), jnp.float32), fb * fb,
                  lax.Precision.HIGHEST)

    iota = lax.broadcasted_iota(jnp.int32, (TN, N), 1)
    lane32 = lax.broadcasted_iota(jnp.int32, (TN, 32), 1)

    def body(j, carry):
        d, idx = carry
        rm = jnp.max(d, axis=1, keepdims=True)
        hit = d == rm
        amf = jnp.min(jnp.where(hit, iota, jnp.int32(N)),
                      axis=1, keepdims=True)
        d = jnp.where(iota == amf, NEG, d)
        idx = jnp.where(lane32 == j, amf + b * N, idx)
        return d, idx

    _, idx = lax.fori_loop(0, KNN, body,
                           (d, jnp.zeros((TN, 32), jnp.int32)))
    idx_ref[0] = idx


def _sc_gather(idxg, f_flat, B, N, C):
    P = 128
    NCH = N // P
    TASKS = (B * NCH) // _NW
    mesh = plsc.VectorSubcoreMesh(core_axis_name="c", subcore_axis_name="s",
                                  num_cores=_NC, num_subcores=_NS)

    @functools.partial(
        pl.kernel, mesh=mesh,
        out_type=jax.ShapeDtypeStruct((B, KNN, N, C), jnp.float32),
        scratch_types=[pltpu.VMEM((P,), jnp.int32),
                       pltpu.VMEM((P, C), jnp.float32),
                       pltpu.SemaphoreType.DMA],
    )
    def k2(idx_hbm, f_hbm, e_hbm, listv, rows, sem):
        wid = lax.axis_index("s") * _NC + lax.axis_index("c")
        for t in range(TASKS):
            g = wid + _NW * t
            b = g // NCH
            n0 = (g % NCH) * P
            for j in range(KNN):
                pltpu.sync_copy(idx_hbm.at[b, j, pl.ds(n0, P)], listv)
                pltpu.async_copy(f_hbm.at[listv], rows, sem).wait()
                pltpu.sync_copy(rows, e_hbm.at[b, j, pl.ds(n0, P), :])

    return k2(idxg, f_flat)


def _conv_body(e_ref, f_ref, w_ref, mx_ref, mn_ref, st_ref, *, TN, C, O):
    b = pl.program_id(0)
    nt = pl.program_id(1)

    @pl.when(jnp.logical_and(b == 0, nt == 0))
    def _():
        st_ref[...] = jnp.zeros_like(st_ref)

    ctr = f_ref[0]
    mx = jnp.full((TN, O), NEG, jnp.float32)
    mn = jnp.full((TN, O), POS, jnp.float32)
    s = jnp.zeros((1, O), jnp.float32)
    s2 = jnp.zeros((1, O), jnp.float32)
    for j in range(KNN):
        feat = jnp.concatenate([e_ref[0, j, :, 0:C] - ctr, ctr],
                               axis=1)
        y = jnp.dot(feat, w_ref[...], preferred_element_type=jnp.float32,
                    precision=lax.Precision.DEFAULT)
        mx = jnp.maximum(mx, y)
        mn = jnp.minimum(mn, y)
        s = s + jnp.sum(y, axis=0, keepdims=True)
        s2 = s2 + jnp.sum(y * y, axis=0, keepdims=True)
    mx_ref[0] = mx
    mn_ref[0] = mn
    st_ref[0:1, :] += s
    st_ref[1:2, :] += s2


def _apply_body(mx_ref, mn_ref, a_ref, c_ref, o_ref):
    a = a_ref[...]
    z = jnp.where(a >= 0, mx_ref[0], mn_ref[0])
    y = a * z + c_ref[...]
    o_ref[0] = jnp.where(y >= 0, y, 0.2 * y)


def _edge_stage(F, W, g, b, B, N, O, C, TN):
    NT = N // TN
    tile = pl.BlockSpec((1, TN, O), lambda bi, ni: (bi, ni, 0))

    TNK = 1024
    idx = pl.pallas_call(
        functools.partial(_knn_idx_body, TN=TNK, N=N, C=C),
        grid=(B, N // TNK),
        in_specs=[pl.BlockSpec((1, N, C), lambda bi, ni: (bi, 0, 0))],
        out_specs=pl.BlockSpec((1, TNK, 32), lambda bi, ni: (bi, ni, 0)),
        out_shape=jax.ShapeDtypeStruct((B, N, 32), jnp.int32),
    )(F)
    idxT = jnp.transpose(idx, (0, 2, 1))[:, :KNN, :]
    CP = 128
    Fp = F if C == CP else jnp.pad(F, ((0, 0), (0, 0), (0, CP - C)))
    f_flat = jnp.reshape(Fp, (B * N, CP))

    NH = N // 2
    NTH = NH // TN
    parts = []
    for h in range(2):
        E = _sc_gather(idxT[:, :, h * NH:(h + 1) * NH], f_flat, B, NH, CP)
        mx, mn, st = pl.pallas_call(
            functools.partial(_conv_body, TN=TN, C=C, O=O),
            grid=(B, NTH),
            in_specs=[pl.BlockSpec((1, KNN, TN, CP),
                                   lambda bi, ni: (bi, 0, ni, 0)),
                      pl.BlockSpec((1, TN, C),
                                   lambda bi, ni, h=h: (bi, ni + h * NTH, 0)),
                      pl.BlockSpec((2 * C, O), lambda bi, ni: (0, 0))],
            out_specs=[pl.BlockSpec((1, TN, O), lambda bi, ni: (bi, ni, 0)),
                       pl.BlockSpec((1, TN, O), lambda bi, ni: (bi, ni, 0)),
                       pl.BlockSpec((8, O), lambda bi, ni: (0, 0))],
            out_shape=[jax.ShapeDtypeStruct((B, NH, O), jnp.float32),
                       jax.ShapeDtypeStruct((B, NH, O), jnp.float32),
                       jax.ShapeDtypeStruct((8, O), jnp.float32)],
        )(E, F, jnp.transpose(W))
        parts.append((mx, mn, st))

    st = parts[0][2] + parts[1][2]
    cnt = B * N * KNN
    mu = st[0] / cnt
    var = st[1] / cnt - mu * mu
    a = g / jnp.sqrt(var + EPS)
    c = b - a * mu

    halves = []
    for h in range(2):
        mx, mn, _ = parts[h]
        halves.append(pl.pallas_call(
            _apply_body,
            grid=(B, NTH),
            in_specs=[pl.BlockSpec((1, TN, O), lambda bi, ni: (bi, ni, 0)),
                      pl.BlockSpec((1, TN, O), lambda bi, ni: (bi, ni, 0)),
                      pl.BlockSpec((1, O), lambda bi, ni: (0, 0)),
                      pl.BlockSpec((1, O), lambda bi, ni: (0, 0))],
            out_specs=pl.BlockSpec((1, TN, O), lambda bi, ni: (bi, ni, 0)),
            out_shape=jax.ShapeDtypeStruct((B, NH, O), jnp.float32),
        )(mx, mn, a[None, :], c[None, :]))
    return jnp.concatenate(halves, axis=1)


def _gram_body(f1_ref, f2_ref, f3_ref, f4_ref, g_ref, s_ref):
    i = pl.program_id(0)
    j = pl.program_id(1)

    @pl.when(jnp.logical_and(i == 0, j == 0))
    def _():
        g_ref[...] = jnp.zeros_like(g_ref)
        s_ref[...] = jnp.zeros_like(s_ref)

    cat = jnp.concatenate(
        [f1_ref[0], f2_ref[0], f3_ref[0], f4_ref[0]], axis=1)
    g_ref[...] += lax.dot_general(cat, cat, (((0,), (0,)), ((), ())),
                                  preferred_element_type=jnp.float32)
    s_ref[0:1, :] += jnp.sum(cat, axis=0, keepdims=True)


def _quad_body(g_ref, w_ref, q_ref):
    wg = jnp.dot(w_ref[...], g_ref[...], preferred_element_type=jnp.float32)
    q_ref[...] = jnp.sum(wg * w_ref[...], axis=1, keepdims=True)


def _head_body(f1_ref, f2_ref, f3_ref, f4_ref, w_ref, a_ref, c_ref, o_ref):
    ni = pl.program_id(1)
    cat = jnp.concatenate(
        [f1_ref[0], f2_ref[0], f3_ref[0], f4_ref[0]], axis=1)
    y = jnp.dot(cat, w_ref[...], preferred_element_type=jnp.float32,
                precision=lax.Precision.DEFAULT)
    y = a_ref[...] * y + c_ref[...]
    z = jnp.where(y >= 0, y, 0.2 * y)
    zmax = jnp.max(z, axis=0, keepdims=True)
    zsum = jnp.sum(z, axis=0, keepdims=True)

    @pl.when(ni == 0)
    def _():
        o_ref[0, 0:1, 0:512] = zmax
        o_ref[0, 0:1, 512:1024] = zsum

    @pl.when(ni != 0)
    def _():
        o_ref[0, 0:1, 0:512] = jnp.maximum(o_ref[0, 0:1, 0:512], zmax)
        o_ref[0, 0:1, 512:1024] += zsum


def _head(F1, F2, F3, F4, W5, g5, b5, B, N, TN):
    NT = N // TN

    def tiles(O):
        return pl.BlockSpec((1, TN, O), lambda bi, ni: (bi, ni, 0))

    G, S = pl.pallas_call(
        _gram_body,
        grid=(B, NT),
        in_specs=[tiles(64), tiles(64), tiles(128), tiles(256)],
        out_specs=[pl.BlockSpec((512, 512), lambda bi, ni: (0, 0)),
                   pl.BlockSpec((8, 512), lambda bi, ni: (0, 0))],
        out_shape=[jax.ShapeDtypeStruct((512, 512), jnp.float32),
                   jax.ShapeDtypeStruct((8, 512), jnp.float32)],
    )(F1, F2, F3, F4)

    q = pl.pallas_call(
        _quad_body,
        out_shape=jax.ShapeDtypeStruct((512, 1), jnp.float32),
    )(G, W5)[:, 0]

    cnt = B * N
    mu = jnp.dot(W5, S[0]) / cnt
    var = q / cnt - mu * mu
    a = g5 / jnp.sqrt(var + EPS)
    c = b5 - a * mu

    out = pl.pallas_call(
        _head_body,
        grid=(B, NT),
        in_specs=[tiles(64), tiles(64), tiles(128), tiles(256),
                  pl.BlockSpec((512, 512), lambda bi, ni: (0, 0)),
                  pl.BlockSpec((1, 512), lambda bi, ni: (0, 0)),
                  pl.BlockSpec((1, 512), lambda bi, ni: (0, 0))],
        out_specs=pl.BlockSpec((1, 8, 1024), lambda bi, ni: (bi, 0, 0)),
        out_shape=jax.ShapeDtypeStruct((B, 8, 1024), jnp.float32),
    )(F1, F2, F3, F4, jnp.transpose(W5), a[None, :], c[None, :])
    out = out[:, 0, :]
    return jnp.concatenate([out[:, :512], out[:, 512:] / N], axis=1)


def kernel(x, W1, W2, W3, W4, W5, g1, b1, g2, b2, g3, b3, g4, b4, g5, b5):
    B, C0, N = x.shape
    TN = 256
    F0 = jnp.transpose(x, (0, 2, 1))
    F1 = _edge_stage(F0, W1, g1, b1, B, N, 64, C0, TN)
    F2 = _edge_stage(F1, W2, g2, b2, B, N, 64, 64, TN)
    F3 = _edge_stage(F2, W3, g3, b3, B, N, 128, 64, TN)
    F4 = _edge_stage(F3, W4, g4, b4, B, N, 256, 128, TN)
    return _head(F1, F2, F3, F4, W5, g5, b5, B, N, TN)


# speedup vs baseline: 7.1394x; 1.0449x over previous
import functools

import jax
import jax.numpy as jnp
from jax import lax
from jax.experimental import pallas as pl
from jax.experimental.pallas import tpu as pltpu
from jax.experimental.pallas import tpu_sc as plsc

KNN = 20
_NC, _NS = 2, 16
_NW = _NC * _NS
EPS = 1e-5
NEG = -3e38
POS = 3e38


def _dotT(a, b, prec):
    return lax.dot_general(a, b, (((1,), (1,)), ((), ())),
                           preferred_element_type=jnp.float32,
                           precision=prec)


def _knn_idx_body(f_ref, idx_ref, *, TN, N, C):
    b = pl.program_id(0)
    nt = pl.program_id(1)
    ft = f_ref[0, pl.ds(nt * TN, TN), :]
    fb = f_ref[0]
    d = 2.0 * _dotT(ft, fb, lax.Precision.DEFAULT)
    d = d - _dotT(jnp.ones((TN, C), jnp.float32), fb * fb,
                  lax.Precision.HIGHEST)

    iota = lax.broadcasted_iota(jnp.int32, (TN, N), 1)
    lane32 = lax.broadcasted_iota(jnp.int32, (TN, 32), 1)

    def body(j, carry):
        d, idx = carry
        rm = jnp.max(d, axis=1, keepdims=True)
        hit = d == rm
        amf = jnp.min(jnp.where(hit, iota, jnp.int32(N)),
                      axis=1, keepdims=True)
        d = jnp.where(iota == amf, NEG, d)
        idx = jnp.where(lane32 == j, amf + b * N, idx)
        return d, idx

    _, idx = lax.fori_loop(0, KNN, body,
                           (d, jnp.zeros((TN, 32), jnp.int32)))
    idx_ref[0] = idx


def _sc_gather(idxg, f_flat, B, N, C):
    P = 128
    NCH = N // P
    TASKS = (B * NCH) // _NW
    mesh = plsc.VectorSubcoreMesh(core_axis_name="c", subcore_axis_name="s",
                                  num_cores=_NC, num_subcores=_NS)

    @functools.partial(
        pl.kernel, mesh=mesh,
        out_type=jax.ShapeDtypeStruct((B, KNN, N, C), jnp.float32),
        scratch_types=[pltpu.VMEM((P,), jnp.int32),
                       pltpu.VMEM((P,), jnp.int32),
                       pltpu.VMEM((P, C), jnp.float32),
                       pltpu.VMEM((P, C), jnp.float32),
                       pltpu.SemaphoreType.DMA,
                       pltpu.SemaphoreType.DMA],
    )
    def k2(idx_hbm, f_hbm, e_hbm, l0, l1, r0, r1, s0, s1):
        wid = lax.axis_index("s") * _NC + lax.axis_index("c")
        bufs = ((l0, r0, s0), (l1, r1, s1))
        seq = [(t, j) for t in range(TASKS) for j in range(KNN)]

        def addr(t, j):
            g = wid + _NW * t
            return g // NCH, (g % NCH) * P, j

        b0, n0, j0 = addr(*seq[0])
        pltpu.sync_copy(idx_hbm.at[b0, j0, pl.ds(n0, P)], l0)
        cur = pltpu.async_copy(f_hbm.at[l0], r0, s0)
        for i, (t, j) in enumerate(seq):
            _, rv, _ = bufs[i % 2]
            if i + 1 < len(seq):
                bn, nn, jn = addr(*seq[i + 1])
                nlv, nrv, nsv = bufs[(i + 1) % 2]
                pltpu.sync_copy(idx_hbm.at[bn, jn, pl.ds(nn, P)], nlv)
                nxt = pltpu.async_copy(f_hbm.at[nlv], nrv, nsv)
            cur.wait()
            b, n0, _ = addr(t, j)
            pltpu.sync_copy(rv, e_hbm.at[b, j, pl.ds(n0, P), :])
            if i + 1 < len(seq):
                cur = nxt

    return k2(idxg, f_flat)


def _conv_body(e_ref, f_ref, w_ref, mx_ref, mn_ref, st_ref, *, TN, C, O):
    b = pl.program_id(0)
    nt = pl.program_id(1)

    @pl.when(jnp.logical_and(b == 0, nt == 0))
    def _():
        st_ref[...] = jnp.zeros_like(st_ref)

    ctr = f_ref[0]
    mx = jnp.full((TN, O), NEG, jnp.float32)
    mn = jnp.full((TN, O), POS, jnp.float32)
    s = jnp.zeros((1, O), jnp.float32)
    s2 = jnp.zeros((1, O), jnp.float32)
    for j in range(KNN):
        feat = jnp.concatenate([e_ref[0, j, :, 0:C] - ctr, ctr],
                               axis=1)
        y = jnp.dot(feat, w_ref[...], preferred_element_type=jnp.float32,
                    precision=lax.Precision.DEFAULT)
        mx = jnp.maximum(mx, y)
        mn = jnp.minimum(mn, y)
        s = s + jnp.sum(y, axis=0, keepdims=True)
        s2 = s2 + jnp.sum(y * y, axis=0, keepdims=True)
    mx_ref[0] = mx
    mn_ref[0] = mn
    st_ref[0:1, :] += s
    st_ref[1:2, :] += s2


def _apply_body(mx_ref, mn_ref, a_ref, c_ref, o_ref):
    a = a_ref[...]
    z = jnp.where(a >= 0, mx_ref[0], mn_ref[0])
    y = a * z + c_ref[...]
    o_ref[0] = jnp.where(y >= 0, y, 0.2 * y)


def _edge_stage(F, W, g, b, B, N, O, C, TN):
    NT = N // TN
    tile = pl.BlockSpec((1, TN, O), lambda bi, ni: (bi, ni, 0))

    TNK = 1024
    idx = pl.pallas_call(
        functools.partial(_knn_idx_body, TN=TNK, N=N, C=C),
        grid=(B, N // TNK),
        in_specs=[pl.BlockSpec((1, N, C), lambda bi, ni: (bi, 0, 0))],
        out_specs=pl.BlockSpec((1, TNK, 32), lambda bi, ni: (bi, ni, 0)),
        out_shape=jax.ShapeDtypeStruct((B, N, 32), jnp.int32),
    )(F)
    idxT = jnp.transpose(idx, (0, 2, 1))[:, :KNN, :]
    CP = 128
    Fp = F if C == CP else jnp.pad(F, ((0, 0), (0, 0), (0, CP - C)))
    f_flat = jnp.reshape(Fp, (B * N, CP))

    NH = N // 2
    NTH = NH // TN
    parts = []
    for h in range(2):
        E = _sc_gather(idxT[:, :, h * NH:(h + 1) * NH], f_flat, B, NH, CP)
        mx, mn, st = pl.pallas_call(
            functools.partial(_conv_body, TN=TN, C=C, O=O),
            grid=(B, NTH),
            in_specs=[pl.BlockSpec((1, KNN, TN, CP),
                                   lambda bi, ni: (bi, 0, ni, 0)),
                      pl.BlockSpec((1, TN, C),
                                   lambda bi, ni, h=h: (bi, ni + h * NTH, 0)),
                      pl.BlockSpec((2 * C, O), lambda bi, ni: (0, 0))],
            out_specs=[pl.BlockSpec((1, TN, O), lambda bi, ni: (bi, ni, 0)),
                       pl.BlockSpec((1, TN, O), lambda bi, ni: (bi, ni, 0)),
                       pl.BlockSpec((8, O), lambda bi, ni: (0, 0))],
            out_shape=[jax.ShapeDtypeStruct((B, NH, O), jnp.float32),
                       jax.ShapeDtypeStruct((B, NH, O), jnp.float32),
                       jax.ShapeDtypeStruct((8, O), jnp.float32)],
        )(E, F, jnp.transpose(W))
        parts.append((mx, mn, st))

    st = parts[0][2] + parts[1][2]
    cnt = B * N * KNN
    mu = st[0] / cnt
    var = st[1] / cnt - mu * mu
    a = g / jnp.sqrt(var + EPS)
    c = b - a * mu

    halves = []
    for h in range(2):
        mx, mn, _ = parts[h]
        halves.append(pl.pallas_call(
            _apply_body,
            grid=(B, NTH),
            in_specs=[pl.BlockSpec((1, TN, O), lambda bi, ni: (bi, ni, 0)),
                      pl.BlockSpec((1, TN, O), lambda bi, ni: (bi, ni, 0)),
                      pl.BlockSpec((1, O), lambda bi, ni: (0, 0)),
                      pl.BlockSpec((1, O), lambda bi, ni: (0, 0))],
            out_specs=pl.BlockSpec((1, TN, O), lambda bi, ni: (bi, ni, 0)),
            out_shape=jax.ShapeDtypeStruct((B, NH, O), jnp.float32),
        )(mx, mn, a[None, :], c[None, :]))
    return jnp.concatenate(halves, axis=1)


def _gram_body(f1_ref, f2_ref, f3_ref, f4_ref, g_ref, s_ref):
    i = pl.program_id(0)
    j = pl.program_id(1)

    @pl.when(jnp.logical_and(i == 0, j == 0))
    def _():
        g_ref[...] = jnp.zeros_like(g_ref)
        s_ref[...] = jnp.zeros_like(s_ref)

    cat = jnp.concatenate(
        [f1_ref[0], f2_ref[0], f3_ref[0], f4_ref[0]], axis=1)
    g_ref[...] += lax.dot_general(cat, cat, (((0,), (0,)), ((), ())),
                                  preferred_element_type=jnp.float32)
    s_ref[0:1, :] += jnp.sum(cat, axis=0, keepdims=True)


def _quad_body(g_ref, w_ref, q_ref):
    wg = jnp.dot(w_ref[...], g_ref[...], preferred_element_type=jnp.float32)
    q_ref[...] = jnp.sum(wg * w_ref[...], axis=1, keepdims=True)


def _head_body(f1_ref, f2_ref, f3_ref, f4_ref, w_ref, a_ref, c_ref, o_ref):
    ni = pl.program_id(1)
    cat = jnp.concatenate(
        [f1_ref[0], f2_ref[0], f3_ref[0], f4_ref[0]], axis=1)
    y = jnp.dot(cat, w_ref[...], preferred_element_type=jnp.float32,
                precision=lax.Precision.DEFAULT)
    y = a_ref[...] * y + c_ref[...]
    z = jnp.where(y >= 0, y, 0.2 * y)
    zmax = jnp.max(z, axis=0, keepdims=True)
    zsum = jnp.sum(z, axis=0, keepdims=True)

    @pl.when(ni == 0)
    def _():
        o_ref[0, 0:1, 0:512] = zmax
        o_ref[0, 0:1, 512:1024] = zsum

    @pl.when(ni != 0)
    def _():
        o_ref[0, 0:1, 0:512] = jnp.maximum(o_ref[0, 0:1, 0:512], zmax)
        o_ref[0, 0:1, 512:1024] += zsum


def _head(F1, F2, F3, F4, W5, g5, b5, B, N, TN):
    NT = N // TN

    def tiles(O):
        return pl.BlockSpec((1, TN, O), lambda bi, ni: (bi, ni, 0))

    G, S = pl.pallas_call(
        _gram_body,
        grid=(B, NT),
        in_specs=[tiles(64), tiles(64), tiles(128), tiles(256)],
        out_specs=[pl.BlockSpec((512, 512), lambda bi, ni: (0, 0)),
                   pl.BlockSpec((8, 512), lambda bi, ni: (0, 0))],
        out_shape=[jax.ShapeDtypeStruct((512, 512), jnp.float32),
                   jax.ShapeDtypeStruct((8, 512), jnp.float32)],
    )(F1, F2, F3, F4)

    q = pl.pallas_call(
        _quad_body,
        out_shape=jax.ShapeDtypeStruct((512, 1), jnp.float32),
    )(G, W5)[:, 0]

    cnt = B * N
    mu = jnp.dot(W5, S[0]) / cnt
    var = q / cnt - mu * mu
    a = g5 / jnp.sqrt(var + EPS)
    c = b5 - a * mu

    out = pl.pallas_call(
        _head_body,
        grid=(B, NT),
        in_specs=[tiles(64), tiles(64), tiles(128), tiles(256),
                  pl.BlockSpec((512, 512), lambda bi, ni: (0, 0)),
                  pl.BlockSpec((1, 512), lambda bi, ni: (0, 0)),
                  pl.BlockSpec((1, 512), lambda bi, ni: (0, 0))],
        out_specs=pl.BlockSpec((1, 8, 1024), lambda bi, ni: (bi, 0, 0)),
        out_shape=jax.ShapeDtypeStruct((B, 8, 1024), jnp.float32),
    )(F1, F2, F3, F4, jnp.transpose(W5), a[None, :], c[None, :])
    out = out[:, 0, :]
    return jnp.concatenate([out[:, :512], out[:, 512:] / N], axis=1)


def kernel(x, W1, W2, W3, W4, W5, g1, b1, g2, b2, g3, b3, g4, b4, g5, b5):
    B, C0, N = x.shape
    TN = 256
    F0 = jnp.transpose(x, (0, 2, 1))
    F1 = _edge_stage(F0, W1, g1, b1, B, N, 64, C0, TN)
    F2 = _edge_stage(F1, W2, g2, b2, B, N, 64, 64, TN)
    F3 = _edge_stage(F2, W3, g3, b3, B, N, 128, 64, TN)
    F4 = _edge_stage(F3, W4, g4, b4, B, N, 256, 128, TN)
    return _head(F1, F2, F3, F4, W5, g5, b5, B, N, TN)
